# baseline (speedup 1.0000x reference)
import jax
import jax.numpy as jnp
from jax.experimental import pallas as pl
from jax.experimental.pallas import tpu as pltpu
from jax.experimental.pallas import tpu_sc as plsc

_CR = 4
_LANES = 16
_N_CORES = 2
_N_SUBCORES = 16


def kernel(x, emb_weight):
    batch, num_patches, dim = x.shape
    n_workers = _N_CORES * _N_SUBCORES
    rows_per_worker = num_patches // n_workers
    n_chunks = rows_per_worker // _CR

    mesh = plsc.VectorSubcoreMesh(core_axis_name="c", subcore_axis_name="s")

    @pl.kernel(
        out_type=jax.ShapeDtypeStruct(x.shape, x.dtype),
        mesh=mesh,
        scratch_types=[
            pltpu.VMEM((2, batch, _CR, dim), x.dtype),
            pltpu.VMEM((2, _CR, dim), x.dtype),
            pltpu.VMEM((2, batch, _CR, dim), x.dtype),
            pltpu.SemaphoreType.DMA((2,)),
            pltpu.SemaphoreType.DMA((2,)),
            pltpu.SemaphoreType.DMA((2,)),
        ],
    )
    def sc_kernel(x_hbm, emb_hbm, o_hbm, xb, eb, ob, sx, se, so):
        c = jax.lax.axis_index("c")
        s = jax.lax.axis_index("s")
        base = (c * _N_SUBCORES + s) * rows_per_worker

        def start_in(k, slot):
            rows = pl.ds(base + k * _CR, _CR)
            pltpu.make_async_copy(
                x_hbm.at[:, rows, :], xb.at[slot], sx.at[slot]
            ).start()
            pltpu.make_async_copy(
                emb_hbm.at[rows, :], eb.at[slot], se.at[slot]
            ).start()

        def wait_in(slot):
            rows = pl.ds(base, _CR)
            pltpu.make_async_copy(
                x_hbm.at[:, rows, :], xb.at[slot], sx.at[slot]
            ).wait()
            pltpu.make_async_copy(
                emb_hbm.at[rows, :], eb.at[slot], se.at[slot]
            ).wait()

        def start_out(k, slot):
            rows = pl.ds(base + k * _CR, _CR)
            pltpu.make_async_copy(
                ob.at[slot], o_hbm.at[:, rows, :], so.at[slot]
            ).start()

        def wait_out(slot):
            rows = pl.ds(base, _CR)
            pltpu.make_async_copy(
                ob.at[slot], o_hbm.at[:, rows, :], so.at[slot]
            ).wait()

        def compute(slot):
            @pl.loop(0, _CR)
            def _(r):
                @pl.loop(0, dim, step=_LANES * 4)
                def _(cc):
                    for u in range(4):
                        cs = pl.ds(cc + u * _LANES, _LANES)
                        e = eb.at[slot, r, cs][...]
                        for b in range(batch):
                            ob.at[slot, b, r, cs][...] = (
                                xb.at[slot, b, r, cs][...] + e
                            )

        start_in(0, 0)
        start_in(1, 1)

        @pl.loop(0, n_chunks, step=2)
        def _(k):
            wait_in(0)

            @pl.when(k > 0)
            def _():
                wait_out(0)

            compute(0)
            start_out(k, 0)

            @pl.when(k + 2 < n_chunks)
            def _():
                start_in(k + 2, 0)

            wait_in(1)

            @pl.when(k > 0)
            def _():
                wait_out(1)

            compute(1)
            start_out(k + 1, 1)

            @pl.when(k + 3 < n_chunks)
            def _():
                start_in(k + 3, 1)

        wait_out(0)
        wait_out(1)

    return sc_kernel(x, emb_weight)


# speedup vs baseline: 3.7840x; 3.7840x over previous
import jax
import jax.numpy as jnp
from jax.experimental import pallas as pl
from jax.experimental.pallas import tpu as pltpu

_BR = 2048


def _add_body(x_ref, emb_ref, out_ref):
    out_ref[0] = x_ref[0] + emb_ref[...]


def kernel(x, emb_weight):
    batch, num_patches, dim = x.shape
    nb = num_patches // _BR
    return pl.pallas_call(
        _add_body,
        grid=(nb, batch),
        in_specs=[
            pl.BlockSpec((1, _BR, dim), lambda i, b: (b, i, 0)),
            pl.BlockSpec((_BR, dim), lambda i, b: (i, 0)),
        ],
        out_specs=pl.BlockSpec((1, _BR, dim), lambda i, b: (b, i, 0)),
        out_shape=jax.ShapeDtypeStruct(x.shape, x.dtype),
        compiler_params=pltpu.CompilerParams(
            dimension_semantics=("parallel", "arbitrary"),
        ),
    )(x, emb_weight)
